# Initial kernel scaffold; baseline (speedup 1.0000x reference)
import jax
import jax.numpy as jnp
from jax.experimental import pallas as pl


def kernel(x):
    raise NotImplementedError("write your pallas kernel here")


# speedup vs baseline: 2.2490x; 2.2490x over previous
import jax
import jax.numpy as jnp
import numpy as np
from jax import lax
from jax.experimental import pallas as pl
from jax.experimental.pallas import tpu as pltpu
from jax.experimental.pallas import tpu_sc as plsc

_ROWS = 128
_N = 32768
_L = 16
_NV = _N // _L
_NC, _NS = 2, 16
_NW = _NC * _NS
_RPW = _ROWS // _NW
_NB = 256
_SIGN = np.int32(-(2 ** 31))


def _sort_body(x_hbm, out_hbm, stage, bufa, bufb, cnt):
    wid = lax.axis_index("s") * _NC + lax.axis_index("c")
    lanes = lax.iota(jnp.int32, _L)
    ones = jnp.ones((_L,), jnp.int32)
    zeros = jnp.zeros((_L,), jnp.int32)
    x_i = x_hbm.bitcast(jnp.int32)
    out_i = out_hbm.bitcast(jnp.int32)

    def do_row(j, _):
        row = wid * _RPW + j
        pltpu.sync_copy(x_i.at[row], stage)

        def xform(v, _):
            ki = stage[pl.ds(v * _L, _L)]
            bufa[pl.ds(v * _L, _L)] = jnp.where(ki < 0, ~ki, ki ^ _SIGN)
            return 0

        lax.fori_loop(0, _NV, xform, 0)

        for p in range(4):
            src = bufa if p % 2 == 0 else bufb
            dst = bufb if p % 2 == 0 else bufa
            sh = 8 * p
            last = p == 3

            def zero(i, _):
                cnt[pl.ds(i * _L, _L)] = zeros
                return 0

            lax.fori_loop(0, _NB, zero, 0)

            def count(v, _):
                d = (src[pl.ds(v * _L, _L)] >> sh) & 0xFF
                plsc.addupdate_scatter(cnt, [d * _L + lanes], ones)
                return 0

            lax.fori_loop(0, _NV, count, 0)

            def scan(i, carry):
                c = cnt[pl.ds(i * _L, _L)]
                cs = plsc.cumsum(c)
                cnt[pl.ds(i * _L, _L)] = carry + cs - c
                return carry + jnp.sum(c)

            lax.fori_loop(0, _NB, scan, jnp.zeros((_L,), jnp.int32))

            def permute(v, _):
                u = src[pl.ds(v * _L, _L)]
                d = (u >> sh) & 0xFF
                idx = d * _L + lanes
                s = plsc.load_gather(cnt, [idx])
                plsc.addupdate_scatter(cnt, [idx], ones)
                if last:
                    orig = jnp.where(u < 0, u ^ _SIGN, ~u)
                    plsc.store_scatter(stage, [s], orig)
                else:
                    phys = ((s & (_NV - 1)) << 4) | (s >> 11)
                    plsc.store_scatter(dst, [phys], u)
                return 0

            lax.fori_loop(0, _NV, permute, 0)

        pltpu.sync_copy(stage, out_i.at[row])
        return 0

    lax.fori_loop(0, _RPW, do_row, 0)


def kernel(x):
    mesh = plsc.VectorSubcoreMesh(core_axis_name="c", subcore_axis_name="s")
    f = pl.kernel(
        _sort_body,
        out_type=jax.ShapeDtypeStruct((_ROWS, _N), jnp.float32),
        mesh=mesh,
        compiler_params=pltpu.CompilerParams(needs_layout_passes=False),
        scratch_types=[
            pltpu.VMEM((_N,), jnp.int32),
            pltpu.VMEM((_N,), jnp.int32),
            pltpu.VMEM((_N,), jnp.int32),
            pltpu.VMEM((_NB * _L,), jnp.int32),
        ],
    )
    return f(x)


# speedup vs baseline: 2.4576x; 1.0928x over previous
import jax
import jax.numpy as jnp
import numpy as np
from jax import lax
from jax.experimental import pallas as pl
from jax.experimental.pallas import tpu as pltpu
from jax.experimental.pallas import tpu_sc as plsc

_ROWS = 128
_N = 32768
_L = 16
_NV = _N // _L
_NQ = 4
_QV = _NV // _NQ
_NC, _NS = 2, 16
_NW = _NC * _NS
_RPW = _ROWS // _NW
_NB = 256
_SIGN = np.int32(-(2 ** 31))


def _fwd(ki):
    return jnp.where(ki < 0, ~ki, ki ^ _SIGN)


def _inv(u):
    return jnp.where(u < 0, u ^ _SIGN, ~u)


def _sort_body(x_hbm, out_hbm, bufa, bufb, c0, c1, c2, c3):
    wid = lax.axis_index("s") * _NC + lax.axis_index("c")
    lanes = lax.iota(jnp.int32, _L)
    ones = jnp.ones((_L,), jnp.int32)
    zeros = jnp.zeros((_L,), jnp.int32)
    x_i = x_hbm.bitcast(jnp.int32)
    out_i = out_hbm.bitcast(jnp.int32)
    cnts = (c0, c1, c2, c3)

    def do_row(j, _):
        row = wid * _RPW + j
        pltpu.sync_copy(x_i.at[row], bufa)

        for p in range(4):
            src = bufa if p % 2 == 0 else bufb
            dst = bufb if p % 2 == 0 else bufa
            sh = 8 * p
            first = p == 0
            last = p == 3

            def zero(i, _):
                for c in cnts:
                    c[pl.ds(i * _L, _L)] = zeros
                return 0

            lax.fori_loop(0, _NB, zero, 0)

            def count(v, _):
                for q, c in enumerate(cnts):
                    u = src[pl.ds((q * _QV + v) * _L, _L)]
                    if first:
                        u = _fwd(u)
                    d = (u >> sh) & 0xFF
                    plsc.addupdate_scatter(c, [d * _L + lanes], ones)
                return 0

            lax.fori_loop(0, _QV, count, 0)

            def scan(i, carry):
                sl = pl.ds(i * _L, _L)
                h0, h1, h2, h3 = c0[sl], c1[sl], c2[sl], c3[sl]
                tot = (h0 + h1) + (h2 + h3)
                e0 = carry + plsc.cumsum(tot) - tot
                e1 = e0 + h0
                e2 = e1 + h1
                e3 = e2 + h2
                c0[sl], c1[sl], c2[sl], c3[sl] = e0, e1, e2, e3
                return carry + jnp.sum(tot)

            lax.fori_loop(0, _NB, scan, jnp.zeros((_L,), jnp.int32))

            def permute(v, _):
                for q, c in enumerate(cnts):
                    u = src[pl.ds((q * _QV + v) * _L, _L)]
                    if first:
                        u = _fwd(u)
                    d = (u >> sh) & 0xFF
                    idx = d * _L + lanes
                    s = plsc.load_gather(c, [idx])
                    plsc.addupdate_scatter(c, [idx], ones)
                    if last:
                        plsc.store_scatter(dst, [s], _inv(u))
                    else:
                        phys = ((s & (_NV - 1)) << 4) | (s >> 11)
                        plsc.store_scatter(dst, [phys], u)
                return 0

            lax.fori_loop(0, _QV, permute, 0)

        pltpu.sync_copy(bufa, out_i.at[row])
        return 0

    lax.fori_loop(0, _RPW, do_row, 0)


def kernel(x):
    mesh = plsc.VectorSubcoreMesh(core_axis_name="c", subcore_axis_name="s")
    f = pl.kernel(
        _sort_body,
        out_type=jax.ShapeDtypeStruct((_ROWS, _N), jnp.float32),
        mesh=mesh,
        compiler_params=pltpu.CompilerParams(needs_layout_passes=False),
        scratch_types=[
            pltpu.VMEM((_N,), jnp.int32),
            pltpu.VMEM((_N,), jnp.int32),
        ] + [
            pltpu.VMEM((_NB * _L,), jnp.int32)
            for _ in range(_NQ)
        ],
    )
    return f(x)
